# baseline (speedup 1.0000x reference)
import jax
import jax.numpy as jnp
from jax import lax
from jax.experimental import pallas as pl
from jax.experimental.pallas import tpu as pltpu

NUM_EXPERTS = 8
TOP_K = 2
DIM = 768
T = 2048
NC = 3
CW = DIM // NC
HW = CW // 2


def _moe_body(x_ref, wr_ref, br_ref, wea_ref, web_ref, be_ref,
              out_ref, aux_ref, gate_ref):
    c = pl.program_id(0)
    e = pl.program_id(1)

    @pl.when((e == 0) & (c == 0))
    def _router():
        x = x_ref[...]
        logits = jnp.dot(x, wr_ref[...],
                         preferred_element_type=jnp.float32) + br_ref[...]
        mx = jnp.max(logits, axis=1, keepdims=True)
        ex = jnp.exp(logits - mx)
        probs = ex / jnp.sum(ex, axis=1, keepdims=True)

        iota = lax.broadcasted_iota(jnp.int32, (T, NUM_EXPERTS), 1)
        m1 = jnp.max(probs, axis=1, keepdims=True)
        a1 = jnp.min(jnp.where(probs == m1, iota, NUM_EXPERTS), axis=1,
                     keepdims=True)
        sel1 = iota == a1
        probs_rest = jnp.where(sel1, -1.0, probs)
        m2 = jnp.max(probs_rest, axis=1, keepdims=True)
        a2 = jnp.min(jnp.where(probs_rest == m2, iota, NUM_EXPERTS), axis=1,
                     keepdims=True)
        sel2 = iota == a2

        gate = jnp.where(sel1, m1, 0.0) + jnp.where(sel2, m2, 0.0)
        gate_ref[...] = gate

        f = jnp.sum(sel1.astype(jnp.float32) + sel2.astype(jnp.float32),
                    axis=0) / (T * TOP_K)
        P = jnp.mean(probs, axis=0)
        aux_ref[0, 0] = NUM_EXPERTS * jnp.sum(f * P)

    @pl.when(e == 0)
    def _bias():
        out_ref[...] = jnp.dot(gate_ref[...], be_ref[0],
                               preferred_element_type=jnp.float32)

    iota = lax.broadcasted_iota(jnp.int32, (T, NUM_EXPERTS), 1)
    g_e = jnp.sum(jnp.where(iota == e, gate_ref[...], 0.0), axis=1,
                  keepdims=True)
    x = x_ref[...]
    out_ref[:, :HW] += g_e * jnp.dot(x, wea_ref[0],
                                     preferred_element_type=jnp.float32)
    out_ref[:, HW:] += g_e * jnp.dot(x, web_ref[0],
                                     preferred_element_type=jnp.float32)


@jax.jit
def kernel(X, Wr, br, We, be):
    Xf = X.reshape(T, DIM)
    br2 = br.reshape(1, NUM_EXPERTS)
    be3 = be.reshape(1, NUM_EXPERTS, DIM)

    out, aux = pl.pallas_call(
        _moe_body,
        grid=(NC, NUM_EXPERTS),
        in_specs=[
            pl.BlockSpec((T, DIM), lambda c, e: (0, 0)),
            pl.BlockSpec((DIM, NUM_EXPERTS), lambda c, e: (0, 0)),
            pl.BlockSpec((1, NUM_EXPERTS), lambda c, e: (0, 0)),
            pl.BlockSpec((1, DIM, HW), lambda c, e: (e, 0, 2 * c)),
            pl.BlockSpec((1, DIM, HW), lambda c, e: (e, 0, 2 * c + 1)),
            pl.BlockSpec((1, NUM_EXPERTS, CW), lambda c, e: (0, 0, c)),
        ],
        out_specs=[
            pl.BlockSpec((T, CW), lambda c, e: (0, c)),
            pl.BlockSpec(memory_space=pltpu.SMEM),
        ],
        out_shape=[
            jax.ShapeDtypeStruct((T, DIM), jnp.float32),
            jax.ShapeDtypeStruct((1, 1), jnp.float32),
        ],
        scratch_shapes=[pltpu.VMEM((T, NUM_EXPERTS), jnp.float32)],
    )(Xf, Wr, br2, We, We, be3)

    return out.reshape(X.shape), aux[0, 0]


# speedup vs baseline: 1.2295x; 1.2295x over previous
import jax
import jax.numpy as jnp
from jax import lax
from jax.experimental import pallas as pl
from jax.experimental.pallas import tpu as pltpu

NUM_EXPERTS = 8
TOP_K = 2
DIM = 768
T = 2048
KBIG = NUM_EXPERTS * DIM
NC = 3
CW = DIM // NC
TC = 128
NCH = T // TC


def _moe_body(x_hbm, wr_ref, br_ref, ws_ref, be_ref,
              out_ref, aux_ref, gate_ref, xbig_ref, xchunk_ref, sem):
    c = pl.program_id(0)

    @pl.when(c == 0)
    def _router():
        iota = lax.broadcasted_iota(jnp.int32, (TC, NUM_EXPERTS), 1)

        def _copy(i, slot):
            return pltpu.make_async_copy(
                x_hbm.at[pl.ds(i * TC, TC), :], xchunk_ref.at[slot],
                sem.at[slot])

        _copy(0, 0).start()

        def _chunk(i, carry):
            fsum, psum = carry
            slot = lax.rem(i, 2)

            @pl.when(i + 1 < NCH)
            def _prefetch():
                _copy(i + 1, lax.rem(i + 1, 2)).start()

            _copy(i, slot).wait()
            xs = xchunk_ref[slot]
            logits = jnp.dot(xs, wr_ref[...],
                             preferred_element_type=jnp.float32) + br_ref[...]
            mx = jnp.max(logits, axis=1, keepdims=True)
            ex = jnp.exp(logits - mx)
            probs = ex / jnp.sum(ex, axis=1, keepdims=True)

            m1 = jnp.max(probs, axis=1, keepdims=True)
            a1 = jnp.min(jnp.where(probs == m1, iota, NUM_EXPERTS), axis=1,
                         keepdims=True)
            sel1 = iota == a1
            probs_rest = jnp.where(sel1, -1.0, probs)
            m2 = jnp.max(probs_rest, axis=1, keepdims=True)
            a2 = jnp.min(jnp.where(probs_rest == m2, iota, NUM_EXPERTS),
                         axis=1, keepdims=True)
            sel2 = iota == a2

            gate = jnp.where(sel1, m1, 0.0) + jnp.where(sel2, m2, 0.0)
            gate_ref[pl.ds(i * TC, TC), :] = gate.astype(jnp.bfloat16)

            for e in range(NUM_EXPERTS):
                g_e = jnp.sum(jnp.where(iota == e, gate, 0.0), axis=1,
                              keepdims=True)
                xbig_ref[pl.ds(i * TC, TC), e * DIM:(e + 1) * DIM] = (
                    g_e * xs).astype(jnp.bfloat16)

            fsum = fsum + jnp.sum(
                sel1.astype(jnp.float32) + sel2.astype(jnp.float32),
                axis=0, keepdims=True)
            psum = psum + jnp.sum(probs, axis=0, keepdims=True)
            return fsum, psum

        z = jnp.zeros((1, NUM_EXPERTS), jnp.float32)
        fsum, psum = lax.fori_loop(0, NCH, _chunk, (z, z))
        f = fsum / (T * TOP_K)
        P = psum / T
        aux_ref[0, 0] = NUM_EXPERTS * jnp.sum(f * P)

    w = ws_ref[...].astype(jnp.bfloat16)
    acc = jnp.dot(xbig_ref[...], w, preferred_element_type=jnp.float32)
    acc += jnp.dot(gate_ref[...], be_ref[...].astype(jnp.bfloat16),
                   preferred_element_type=jnp.float32)
    out_ref[...] = acc


@jax.jit
def kernel(X, Wr, br, We, be):
    Xf = X.reshape(T, DIM)
    br2 = br.reshape(1, NUM_EXPERTS)
    Ws = We.reshape(KBIG, DIM)

    out, aux = pl.pallas_call(
        _moe_body,
        grid=(NC,),
        in_specs=[
            pl.BlockSpec(memory_space=pl.ANY),
            pl.BlockSpec((DIM, NUM_EXPERTS), lambda c: (0, 0)),
            pl.BlockSpec((1, NUM_EXPERTS), lambda c: (0, 0)),
            pl.BlockSpec((KBIG, CW), lambda c: (0, c)),
            pl.BlockSpec((NUM_EXPERTS, CW), lambda c: (0, c)),
        ],
        out_specs=[
            pl.BlockSpec((T, CW), lambda c: (0, c)),
            pl.BlockSpec(memory_space=pltpu.SMEM),
        ],
        out_shape=[
            jax.ShapeDtypeStruct((T, DIM), jnp.float32),
            jax.ShapeDtypeStruct((1, 1), jnp.float32),
        ],
        scratch_shapes=[
            pltpu.VMEM((T, NUM_EXPERTS), jnp.bfloat16),
            pltpu.VMEM((T, KBIG), jnp.bfloat16),
            pltpu.VMEM((2, TC, DIM), jnp.float32),
            pltpu.SemaphoreType.DMA((2,)),
        ],
    )(Xf, Wr, br2, Ws, be)

    return out.reshape(X.shape), aux[0, 0]


# speedup vs baseline: 1.4686x; 1.1944x over previous
import jax
import jax.numpy as jnp
from jax import lax
from jax.experimental import pallas as pl
from jax.experimental.pallas import tpu as pltpu

NUM_EXPERTS = 8
TOP_K = 2
DIM = 768
T = 2048
KBIG = NUM_EXPERTS * DIM
NC = 3
CW = DIM // NC
TH = 1024
TB = 512
NB = TH // TB


def _router_body(x_ref, wr_ref, br_ref, gate_ref, aux_ref):
    x = x_ref[...]
    logits = jnp.dot(x, wr_ref[...],
                     preferred_element_type=jnp.float32) + br_ref[...]
    mx = jnp.max(logits, axis=1, keepdims=True)
    ex = jnp.exp(logits - mx)
    probs = ex / jnp.sum(ex, axis=1, keepdims=True)

    iota = lax.broadcasted_iota(jnp.int32, (T, NUM_EXPERTS), 1)
    m1 = jnp.max(probs, axis=1, keepdims=True)
    a1 = jnp.min(jnp.where(probs == m1, iota, NUM_EXPERTS), axis=1,
                 keepdims=True)
    sel1 = iota == a1
    probs_rest = jnp.where(sel1, -1.0, probs)
    m2 = jnp.max(probs_rest, axis=1, keepdims=True)
    a2 = jnp.min(jnp.where(probs_rest == m2, iota, NUM_EXPERTS), axis=1,
                 keepdims=True)
    sel2 = iota == a2

    gate_ref[...] = jnp.where(sel1, m1, 0.0) + jnp.where(sel2, m2, 0.0)

    f = jnp.sum(sel1.astype(jnp.float32) + sel2.astype(jnp.float32),
                axis=0) / (T * TOP_K)
    P = jnp.mean(probs, axis=0)
    aux_ref[0, 0] = NUM_EXPERTS * jnp.sum(f * P)


def _moe_body(x_ref, gate_ref, ws_ref, be_ref, out_ref, xbig_ref):
    c = pl.program_id(1)

    @pl.when(c == 0)
    def _build():
        iota = lax.broadcasted_iota(jnp.int32, (TB, NUM_EXPERTS), 1)

        def _chunk(i, carry):
            xs = x_ref[pl.ds(i * TB, TB), :]
            gs = gate_ref[pl.ds(i * TB, TB), :]
            for e in range(NUM_EXPERTS):
                g_e = jnp.sum(jnp.where(iota == e, gs, 0.0), axis=1,
                              keepdims=True)
                xbig_ref[pl.ds(i * TB, TB), e * DIM:(e + 1) * DIM] = (
                    g_e * xs).astype(jnp.bfloat16)
            return carry

        lax.fori_loop(0, NB, _chunk, 0)

    w = ws_ref[...].astype(jnp.bfloat16)
    acc = jnp.dot(xbig_ref[...], w, preferred_element_type=jnp.float32)
    acc += jnp.dot(gate_ref[...].astype(jnp.bfloat16),
                   be_ref[...].astype(jnp.bfloat16),
                   preferred_element_type=jnp.float32)
    out_ref[...] = acc


@jax.jit
def kernel(X, Wr, br, We, be):
    Xf = X.reshape(T, DIM)
    br2 = br.reshape(1, NUM_EXPERTS)
    Ws = We.reshape(KBIG, DIM)

    gate, aux = pl.pallas_call(
        _router_body,
        in_specs=[
            pl.BlockSpec((T, DIM), lambda: (0, 0)),
            pl.BlockSpec((DIM, NUM_EXPERTS), lambda: (0, 0)),
            pl.BlockSpec((1, NUM_EXPERTS), lambda: (0, 0)),
        ],
        out_specs=[
            pl.BlockSpec((T, NUM_EXPERTS), lambda: (0, 0)),
            pl.BlockSpec(memory_space=pltpu.SMEM),
        ],
        out_shape=[
            jax.ShapeDtypeStruct((T, NUM_EXPERTS), jnp.float32),
            jax.ShapeDtypeStruct((1, 1), jnp.float32),
        ],
    )(Xf, Wr, br2)

    out = pl.pallas_call(
        _moe_body,
        grid=(T // TH, NC),
        in_specs=[
            pl.BlockSpec((TH, DIM), lambda h, c: (h, 0)),
            pl.BlockSpec((TH, NUM_EXPERTS), lambda h, c: (h, 0)),
            pl.BlockSpec((KBIG, CW), lambda h, c: (0, c)),
            pl.BlockSpec((NUM_EXPERTS, CW), lambda h, c: (0, c)),
        ],
        out_specs=pl.BlockSpec((TH, CW), lambda h, c: (h, c)),
        out_shape=jax.ShapeDtypeStruct((T, DIM), jnp.float32),
        scratch_shapes=[
            pltpu.VMEM((TH, KBIG), jnp.bfloat16),
        ],
    )(Xf, gate, Ws, be)

    return out.reshape(X.shape), aux[0, 0]
